# baseline (speedup 1.0000x reference)
import jax
import jax.numpy as jnp
from jax import lax
from jax.experimental import pallas as pl
from jax.experimental.pallas import tpu as pltpu
from jax.experimental.pallas import tpu_sc as plsc

N_U = 10000
N_I = 10000
D = 128
NNZ = 320000

NUM_CORES = 2
NUM_TILES = 16
LANES = 16
EDGES_PER_TILE = NNZ // NUM_TILES
K = 40
CHUNKS = EDGES_PER_TILE // K
NSLOT = 2
ROW_BLOCK = 640
CH = 80
D_VECS = D // LANES


def _sc_body(table, erows, ecols, evals, out, *rest):
  acc = rest[0]
  groups = [rest[1 + k * NSLOT:1 + (k + 1) * NSLOT] for k in range(9)]
  (cols_b, rows_b, vals_b, gbuf,
   sem_c, sem_r, sem_v, sem_g, sem_s) = groups

  cid = lax.axis_index("c")
  tid = lax.axis_index("s")
  row_base = tid * ROW_BLOCK
  n_row_chunks = jnp.where(tid == NUM_TILES - 1, 5, ROW_BLOCK // CH)

  zero = jnp.zeros((LANES,), jnp.float32)

  def zero_row(j, carry):
    for g in range(D_VECS):
      gbuf[0][j, pl.ds(LANES * g, LANES)] = zero
    return carry

  lax.fori_loop(0, CH, zero_row, 0)

  def zero_chunk(j, carry):
    pltpu.sync_copy(gbuf[0].at[pl.ds(0, CH), :], acc.at[pl.ds(row_base + j * CH, CH), :])
    return carry

  lax.fori_loop(0, n_row_chunks, zero_chunk, 0)

  plsc.subcore_barrier()

  tile_base = cid * NNZ + tid * EDGES_PER_TILE

  def fetch(ref, i, dst, sem):
    pltpu.async_copy(ref.at[pl.ds(tile_base + i * K, K)], dst, sem)

  def wait_i32(dst, sem):
    pltpu.make_async_copy(erows.at[pl.ds(0, K)], dst, sem).wait()

  def wait_f32(dst, sem):
    pltpu.make_async_copy(evals.at[pl.ds(0, K)], dst, sem).wait()

  def wait_scatter(s):
    pltpu.make_async_copy(gbuf[s], acc.at[rows_b[s]], sem_s[s]).wait()

  def issue_gather(s):
    pltpu.async_copy(table.at[cols_b[s]], gbuf[s], sem_g[s])

  def wait_gather(s):
    pltpu.make_async_copy(table.at[cols_b[s]], gbuf[s], sem_g[s]).wait()

  def scale(s):
    nq = (K + LANES - 1) // LANES
    offs = [min(LANES * q, K - LANES) for q in range(nq)]
    vchunks = [vals_b[s][pl.ds(o, LANES)] for o in offs]
    for e in range(K):
      q = min(e // LANES, nq - 1)
      val = jnp.broadcast_to(vchunks[q][e - offs[q]], (LANES,))
      for g in range(D_VECS):
        sl = pl.ds(LANES * g, LANES)
        gbuf[s][e, sl] = gbuf[s][e, sl] * val

  def finish_chunk(i, s, prefetch_vals):
    wait_f32(vals_b[s], sem_v[s])
    scale(s)
    if prefetch_vals:
      @pl.when(i + NSLOT < CHUNKS)
      def _():
        fetch(evals, i + NSLOT, vals_b[s], sem_v[s])
    wait_i32(rows_b[s], sem_r[s])
    pltpu.async_copy(gbuf[s], acc.at[rows_b[s]], sem_s[s], add=True)

  for s in range(NSLOT):
    fetch(ecols, s, cols_b[s], sem_c[s])
    fetch(evals, s, vals_b[s], sem_v[s])
  for s in range(NSLOT - 1):
    fetch(erows, s, rows_b[s], sem_r[s])
  for s in range(NSLOT - 1):
    wait_i32(cols_b[s], sem_c[s])
    issue_gather(s)

  def chunk(i, s):
    wait_gather(s)

    @pl.when(i + NSLOT < CHUNKS)
    def _():
      fetch(ecols, i + NSLOT, cols_b[s], sem_c[s])

    sp = (s + NSLOT - 1) % NSLOT

    @pl.when(i + NSLOT - 1 < CHUNKS)
    def _():
      wait_i32(cols_b[sp], sem_c[sp])

      @pl.when(i >= 1)
      def _():
        wait_scatter(sp)

      issue_gather(sp)
      fetch(erows, i + NSLOT - 1, rows_b[sp], sem_r[sp])

    finish_chunk(i, s, True)

  def quad(j, carry):
    for s in range(NSLOT):
      chunk(NSLOT * j + s, s)
    return carry

  n_quads = CHUNKS // NSLOT
  lax.fori_loop(0, n_quads, quad, 0)

  for s in range(CHUNKS - NSLOT * n_quads):
    wait_gather(s)
    finish_chunk(NSLOT * n_quads + s, s, False)
  for s in range(NSLOT):
    wait_scatter(s)

  plsc.subcore_barrier()

  def out_chunk(j, carry):
    off = row_base + j * CH
    pltpu.sync_copy(acc.at[pl.ds(off, CH), :], gbuf[0].at[pl.ds(0, CH), :])
    pltpu.sync_copy(gbuf[0].at[pl.ds(0, CH), :], out.at[pl.ds(cid * N_U + off, CH), :])
    return carry

  lax.fori_loop(0, n_row_chunks, out_chunk, 0)


@jax.jit
def _spmm_sc(table, erows, ecols, evals):
  mesh = plsc.VectorSubcoreMesh(core_axis_name="c", subcore_axis_name="s")
  scratch = (
      [pltpu.VMEM_SHARED((N_U, D), jnp.float32)]
      + [pltpu.VMEM((K,), jnp.int32) for _ in range(2 * NSLOT)]
      + [pltpu.VMEM((K,), jnp.float32) for _ in range(NSLOT)]
      + [pltpu.VMEM((K, D), jnp.float32) for _ in range(NSLOT)]
      + [pltpu.SemaphoreType.DMA for _ in range(5 * NSLOT)]
  )
  return pl.kernel(
      _sc_body,
      out_type=jax.ShapeDtypeStruct((N_U + N_I, D), jnp.float32),
      mesh=mesh,
      scratch_types=scratch,
  )(table, erows, ecols, evals)


def kernel(input, user_indices, user_values, item_indices, item_values):
  erows = jnp.concatenate([user_indices[0], item_indices[0]])
  ecols = jnp.concatenate([user_indices[1], item_indices[1] + N_U])
  evals = jnp.concatenate([user_values, item_values])
  return _spmm_sc(input, erows, ecols, evals)


# speedup vs baseline: 1.4374x; 1.4374x over previous
import jax
import jax.numpy as jnp
from jax import lax
from jax.experimental import pallas as pl
from jax.experimental.pallas import tpu as pltpu
from jax.experimental.pallas import tpu_sc as plsc

N_U = 10000
N_I = 10000
D = 128
NNZ = 320000

NUM_CORES = 2
NUM_TILES = 16
LANES = 16
EDGES_PER_TILE = NNZ // NUM_TILES
K = 80
CHUNKS = EDGES_PER_TILE // K
NSLOT = 2
ROW_BLOCK = 640
CH = 80
D_VECS = D // LANES


def _sc_body(table, erows, ecols, evals, out, *rest):
  acc = rest[0]
  groups = [rest[1 + k * NSLOT:1 + (k + 1) * NSLOT] for k in range(9)]
  (cols_b, rows_b, vals_b, gbuf,
   sem_c, sem_r, sem_v, sem_g, sem_s) = groups

  cid = lax.axis_index("c")
  tid = lax.axis_index("s")
  row_base = tid * ROW_BLOCK
  n_row_chunks = jnp.where(tid == NUM_TILES - 1, 5, ROW_BLOCK // CH)

  zero = jnp.zeros((LANES,), jnp.float32)

  def zero_row(j, carry):
    for g in range(D_VECS):
      gbuf[0][j, pl.ds(LANES * g, LANES)] = zero
    return carry

  lax.fori_loop(0, CH, zero_row, 0)

  def zero_chunk(j, carry):
    pltpu.sync_copy(gbuf[0].at[pl.ds(0, CH), :], acc.at[pl.ds(row_base + j * CH, CH), :])
    return carry

  lax.fori_loop(0, n_row_chunks, zero_chunk, 0)

  plsc.subcore_barrier()

  tile_base = cid * NNZ + tid * EDGES_PER_TILE

  def fetch(ref, i, dst, sem):
    pltpu.async_copy(ref.at[pl.ds(tile_base + i * K, K)], dst, sem)

  def wait_i32(dst, sem):
    pltpu.make_async_copy(erows.at[pl.ds(0, K)], dst, sem).wait()

  def wait_f32(dst, sem):
    pltpu.make_async_copy(evals.at[pl.ds(0, K)], dst, sem).wait()

  def wait_scatter(s):
    pltpu.make_async_copy(gbuf[s], acc.at[rows_b[s]], sem_s[s]).wait()

  def issue_gather(s):
    pltpu.async_copy(table.at[cols_b[s]], gbuf[s], sem_g[s])

  def wait_gather(s):
    pltpu.make_async_copy(table.at[cols_b[s]], gbuf[s], sem_g[s]).wait()

  def scale(s):
    nq = (K + LANES - 1) // LANES
    offs = [min(LANES * q, K - LANES) for q in range(nq)]
    vchunks = [vals_b[s][pl.ds(o, LANES)] for o in offs]
    for e in range(K):
      q = min(e // LANES, nq - 1)
      val = jnp.broadcast_to(vchunks[q][e - offs[q]], (LANES,))
      for g in range(D_VECS):
        sl = pl.ds(LANES * g, LANES)
        gbuf[s][e, sl] = gbuf[s][e, sl] * val

  def finish_chunk(i, s, prefetch_vals):
    wait_f32(vals_b[s], sem_v[s])
    scale(s)
    if prefetch_vals:
      @pl.when(i + NSLOT < CHUNKS)
      def _():
        fetch(evals, i + NSLOT, vals_b[s], sem_v[s])
    wait_i32(rows_b[s], sem_r[s])
    pltpu.async_copy(gbuf[s], acc.at[rows_b[s]], sem_s[s], add=True)

  for s in range(NSLOT):
    fetch(ecols, s, cols_b[s], sem_c[s])
    fetch(evals, s, vals_b[s], sem_v[s])
  for s in range(NSLOT - 1):
    fetch(erows, s, rows_b[s], sem_r[s])
  for s in range(NSLOT - 1):
    wait_i32(cols_b[s], sem_c[s])
    issue_gather(s)

  def chunk(i, s):
    wait_gather(s)

    @pl.when(i + NSLOT < CHUNKS)
    def _():
      fetch(ecols, i + NSLOT, cols_b[s], sem_c[s])

    sp = (s + NSLOT - 1) % NSLOT

    @pl.when(i + NSLOT - 1 < CHUNKS)
    def _():
      wait_i32(cols_b[sp], sem_c[sp])

      @pl.when(i >= 1)
      def _():
        wait_scatter(sp)

      issue_gather(sp)
      fetch(erows, i + NSLOT - 1, rows_b[sp], sem_r[sp])

    finish_chunk(i, s, True)

  def quad(j, carry):
    for s in range(NSLOT):
      chunk(NSLOT * j + s, s)
    return carry

  n_quads = CHUNKS // NSLOT
  lax.fori_loop(0, n_quads, quad, 0)

  for s in range(CHUNKS - NSLOT * n_quads):
    wait_gather(s)
    finish_chunk(NSLOT * n_quads + s, s, False)
  for s in range(NSLOT):
    wait_scatter(s)

  plsc.subcore_barrier()

  def out_chunk(j, carry):
    off = row_base + j * CH
    pltpu.sync_copy(acc.at[pl.ds(off, CH), :], gbuf[0].at[pl.ds(0, CH), :])
    pltpu.sync_copy(gbuf[0].at[pl.ds(0, CH), :], out.at[pl.ds(cid * N_U + off, CH), :])
    return carry

  lax.fori_loop(0, n_row_chunks, out_chunk, 0)


@jax.jit
def _spmm_sc(table, erows, ecols, evals):
  mesh = plsc.VectorSubcoreMesh(core_axis_name="c", subcore_axis_name="s")
  scratch = (
      [pltpu.VMEM_SHARED((N_U, D), jnp.float32)]
      + [pltpu.VMEM((K,), jnp.int32) for _ in range(2 * NSLOT)]
      + [pltpu.VMEM((K,), jnp.float32) for _ in range(NSLOT)]
      + [pltpu.VMEM((K, D), jnp.float32) for _ in range(NSLOT)]
      + [pltpu.SemaphoreType.DMA for _ in range(5 * NSLOT)]
  )
  return pl.kernel(
      _sc_body,
      out_type=jax.ShapeDtypeStruct((N_U + N_I, D), jnp.float32),
      mesh=mesh,
      scratch_types=scratch,
  )(table, erows, ecols, evals)


def kernel(input, user_indices, user_values, item_indices, item_values):
  erows = jnp.concatenate([user_indices[0], item_indices[0]])
  ecols = jnp.concatenate([user_indices[1], item_indices[1] + N_U])
  evals = jnp.concatenate([user_values, item_values])
  return _spmm_sc(input, erows, ecols, evals)
